# baseline (speedup 1.0000x reference)
import functools

import jax
import jax.numpy as jnp
from jax import lax
from jax.experimental import pallas as pl
from jax.experimental.pallas import tpu as pltpu
from jax.experimental.pallas import tpu_sc as plsc

B = 16384
F = 26
D = 32
V = 1000012
ED = F * D
EP = 1024
H1, H2, H3 = 512, 256, 128

NC, NS = 2, 16
NW = NC * NS
EPW = B // NW
EPC = 64
NCH = EPW // EPC
GRP = 8


def _sc_body(x, emb, fc, eout, fcg, idx_v, rows_v, fcr_v, sem_e, sem_f):
    w = lax.axis_index("s") * NC + lax.axis_index("c")
    zero16 = jnp.zeros((16,), jnp.float32)
    for e in range(EPC):
        fcr_v[e, pl.ds(16, 16)] = zero16

    def chunk(g, carry):
        base = w * EPW + g * EPC
        pltpu.sync_copy(x.at[pl.ds(base, EPC)], idx_v)

        def grp(gi, carry2):
            cps = []
            for k in range(GRP):
                e = gi * GRP + k
                cps.append(pltpu.async_copy(
                    emb.at[idx_v.at[e]],
                    rows_v.at[e].at[pl.ds(0, F)], sem_e))
                cps.append(pltpu.async_copy(
                    fc.at[idx_v.at[e]],
                    fcr_v.at[e].at[pl.ds(0, F)], sem_f))
            for c in cps:
                c.wait()
            return carry2

        lax.fori_loop(0, EPC // GRP, grp, 0)
        pltpu.sync_copy(rows_v, eout.at[pl.ds(base, EPC)])
        pltpu.sync_copy(fcr_v, fcg.at[pl.ds(base, EPC)])
        return carry

    lax.fori_loop(0, NCH, chunk, 0)


@functools.lru_cache(maxsize=None)
def _make_sc_gather():
    return pl.kernel(
        _sc_body,
        out_type=[
            jax.ShapeDtypeStruct((B, D, D), jnp.float32),
            jax.ShapeDtypeStruct((B, D), jnp.float32),
        ],
        mesh=plsc.VectorSubcoreMesh(
            core_axis_name="c", subcore_axis_name="s",
            num_cores=NC, num_subcores=NS),
        scratch_types=[
            pltpu.VMEM((EPC, F), jnp.int32),
            pltpu.VMEM((EPC, D, D), jnp.float32),
            pltpu.VMEM((EPC, D), jnp.float32),
            pltpu.SemaphoreType.DMA,
            pltpu.SemaphoreType.DMA,
        ],
        compiler_params=pltpu.CompilerParams(use_tc_tiling_on_sc=False),
    )


def _mlp_body(e_ref, fcg_ref, s_ref, w1_ref, b1_ref, w2_ref, b2_ref,
              w3_ref, b3_ref, w4_ref, c_ref, o_ref):
    e = e_ref[...]
    s = jnp.dot(e, s_ref[...], preferred_element_type=jnp.float32)
    q = jnp.dot(e * e, s_ref[...], preferred_element_type=jnp.float32)
    pair = 0.5 * jnp.sum(s * s - q, axis=1, keepdims=True)
    lin = jnp.sum(fcg_ref[...], axis=1, keepdims=True)
    h = jnp.maximum(
        jnp.dot(e, w1_ref[...], preferred_element_type=jnp.float32)
        + b1_ref[...], 0.0)
    h = jnp.maximum(
        jnp.dot(h, w2_ref[...], preferred_element_type=jnp.float32)
        + b2_ref[...], 0.0)
    h = jnp.maximum(
        jnp.dot(h, w3_ref[...], preferred_element_type=jnp.float32)
        + b3_ref[...], 0.0)
    mlp = jnp.sum(h * w4_ref[...], axis=1, keepdims=True)
    z = pair + lin + mlp + c_ref[0, 0]
    o_ref[...] = 1.0 / (1.0 + jnp.exp(-z))


def kernel(x, bias, fc, emb, W1, b1, W2, b2, W3, b3, W4, b4):
    eout, fcg = _make_sc_gather()(x, emb, fc.reshape(-1))
    e2 = eout.reshape(B, EP)
    W1p = jnp.pad(W1, ((0, EP - ED), (0, 0)))
    rows = jnp.arange(EP, dtype=jnp.int32)
    S = ((rows[:, None] % D == jnp.arange(D, dtype=jnp.int32)[None, :])
         & (rows[:, None] < ED)).astype(jnp.float32)
    c = (bias + b4).reshape(1, 1)

    bB = 1024
    grid = (B // bB,)
    out2 = pl.pallas_call(
        _mlp_body,
        grid=grid,
        in_specs=[
            pl.BlockSpec((bB, EP), lambda i: (i, 0)),
            pl.BlockSpec((bB, D), lambda i: (i, 0)),
            pl.BlockSpec((EP, D), lambda i: (0, 0)),
            pl.BlockSpec((EP, H1), lambda i: (0, 0)),
            pl.BlockSpec((1, H1), lambda i: (0, 0)),
            pl.BlockSpec((H1, H2), lambda i: (0, 0)),
            pl.BlockSpec((1, H2), lambda i: (0, 0)),
            pl.BlockSpec((H2, H3), lambda i: (0, 0)),
            pl.BlockSpec((1, H3), lambda i: (0, 0)),
            pl.BlockSpec((1, H3), lambda i: (0, 0)),
            pl.BlockSpec((1, 1), lambda i: (0, 0)),
        ],
        out_specs=pl.BlockSpec((bB, 1), lambda i: (i, 0)),
        out_shape=jax.ShapeDtypeStruct((B, 1), jnp.float32),
    )(e2, fcg, S, W1p, b1.reshape(1, H1), W2, b2.reshape(1, H2),
      W3, b3.reshape(1, H3), W4.reshape(1, H3), c)
    return out2.reshape(B)


# speedup vs baseline: 1.4108x; 1.4108x over previous
import functools

import jax
import jax.numpy as jnp
from jax import lax
from jax.experimental import pallas as pl
from jax.experimental.pallas import tpu as pltpu
from jax.experimental.pallas import tpu_sc as plsc

B = 16384
F = 26
D = 32
V = 1000012
ED = F * D
EP = 1024
H1, H2, H3 = 512, 256, 128

NC, NS = 2, 16
NW = NC * NS
EPW = B // NW
EPC = 64
NCH = EPW // EPC
GRP = 8


IPW = EPW * F
RPC = 8
IPC = RPC * 128
NCHF = IPW // IPC
RPW = IPW // 128


def _sc_body(x3, emb, fc, eout, fcg, idx_v, rows_v, fcr_v, sem_e, sem_f):
    w = lax.axis_index("s") * NC + lax.axis_index("c")

    def chunk(g, carry):
        rowoff = w * RPW + g * RPC
        pltpu.sync_copy(x3.at[pl.ds(rowoff, RPC)], idx_v)
        cps = []
        for j in range(RPC):
            cps.append(pltpu.async_copy(
                emb.at[idx_v.at[j]], rows_v.at[pl.ds(j * 128, 128)], sem_e))
            cps.append(pltpu.async_copy(
                fc.at[idx_v.at[j]], fcr_v.at[pl.ds(j * 128, 128)], sem_f))
        for c in cps:
            c.wait()
        ibase = w * IPW + g * IPC
        pltpu.sync_copy(rows_v, eout.at[pl.ds(ibase, IPC)])
        pltpu.sync_copy(fcr_v, fcg.at[pl.ds(ibase, IPC)])
        return carry

    lax.fori_loop(0, NCHF, chunk, 0)


@functools.lru_cache(maxsize=None)
def _make_sc_gather():
    return pl.kernel(
        _sc_body,
        out_type=[
            jax.ShapeDtypeStruct((B * F, D), jnp.float32),
            jax.ShapeDtypeStruct((B * F,), jnp.float32),
        ],
        mesh=plsc.VectorSubcoreMesh(
            core_axis_name="c", subcore_axis_name="s",
            num_cores=NC, num_subcores=NS),
        scratch_types=[
            pltpu.VMEM((RPC, 128), jnp.int32),
            pltpu.VMEM((IPC, D), jnp.float32),
            pltpu.VMEM((IPC,), jnp.float32),
            pltpu.SemaphoreType.DMA,
            pltpu.SemaphoreType.DMA,
        ],
        compiler_params=pltpu.CompilerParams(use_tc_tiling_on_sc=False),
    )


def _mlp_body(e_ref, fcg_ref, s_ref, w1_ref, b1_ref, w2_ref, b2_ref,
              w3_ref, b3_ref, w4_ref, c_ref, o_ref):
    e = e_ref[...]
    s = jnp.dot(e, s_ref[...], preferred_element_type=jnp.float32)
    q = jnp.dot(e * e, s_ref[...], preferred_element_type=jnp.float32)
    pair = 0.5 * jnp.sum(s * s - q, axis=1, keepdims=True)
    lin = jnp.sum(fcg_ref[...], axis=1, keepdims=True)
    h = jnp.maximum(
        jnp.dot(e, w1_ref[...], preferred_element_type=jnp.float32)
        + b1_ref[...], 0.0)
    h = jnp.maximum(
        jnp.dot(h, w2_ref[...], preferred_element_type=jnp.float32)
        + b2_ref[...], 0.0)
    h = jnp.maximum(
        jnp.dot(h, w3_ref[...], preferred_element_type=jnp.float32)
        + b3_ref[...], 0.0)
    mlp = jnp.sum(h * w4_ref[...], axis=1, keepdims=True)
    z = pair + lin + mlp + c_ref[0, 0]
    o_ref[...] = 1.0 / (1.0 + jnp.exp(-z))


def kernel(x, bias, fc, emb, W1, b1, W2, b2, W3, b3, W4, b4):
    x3 = x.reshape(B * F // 128, 128)
    eflat, fcg = _make_sc_gather()(x3, emb, fc[:, 0])
    e2 = eflat.reshape(B, ED)
    fcg2 = fcg.reshape(B, F)
    rows = jnp.arange(ED, dtype=jnp.int32)
    S = (rows[:, None] % D
         == jnp.arange(D, dtype=jnp.int32)[None, :]).astype(jnp.float32)
    c = (bias + b4).reshape(1, 1)

    bB = 1024
    grid = (B // bB,)
    out2 = pl.pallas_call(
        _mlp_body,
        grid=grid,
        in_specs=[
            pl.BlockSpec((bB, ED), lambda i: (i, 0)),
            pl.BlockSpec((bB, F), lambda i: (i, 0)),
            pl.BlockSpec((ED, D), lambda i: (0, 0)),
            pl.BlockSpec((ED, H1), lambda i: (0, 0)),
            pl.BlockSpec((1, H1), lambda i: (0, 0)),
            pl.BlockSpec((H1, H2), lambda i: (0, 0)),
            pl.BlockSpec((1, H2), lambda i: (0, 0)),
            pl.BlockSpec((H2, H3), lambda i: (0, 0)),
            pl.BlockSpec((1, H3), lambda i: (0, 0)),
            pl.BlockSpec((1, H3), lambda i: (0, 0)),
            pl.BlockSpec((1, 1), lambda i: (0, 0)),
        ],
        out_specs=pl.BlockSpec((bB, 1), lambda i: (i, 0)),
        out_shape=jax.ShapeDtypeStruct((B, 1), jnp.float32),
    )(e2, fcg2, S, W1, b1.reshape(1, H1), W2, b2.reshape(1, H2),
      W3, b3.reshape(1, H3), W4.reshape(1, H3), c)
    return out2.reshape(B)


# speedup vs baseline: 1.4487x; 1.0269x over previous
import functools

import jax
import jax.numpy as jnp
from jax import lax
from jax.experimental import pallas as pl
from jax.experimental.pallas import tpu as pltpu
from jax.experimental.pallas import tpu_sc as plsc

B = 16384
F = 26
D = 32
V = 1000012
ED = F * D
H1, H2, H3 = 512, 256, 128
BF = B * F

NC, NS = 2, 16
NW = NC * NS
EPW = B // NW
IPW = EPW * F
RPC = 8
IPC = RPC * 128
NCH = IPW // IPC
RPW = IPW // 128


def _sc_emb_body(x3, emb, eout, idx_v, rows_v, sem_e):
    w = lax.axis_index("s") * NC + lax.axis_index("c")

    def chunk(g, carry):
        rowoff = w * RPW + g * RPC
        pltpu.sync_copy(x3.at[pl.ds(rowoff, RPC)], idx_v)
        cps = []
        for j in range(RPC):
            cps.append(pltpu.async_copy(
                emb.at[idx_v.at[j]], rows_v.at[pl.ds(j * 128, 128)], sem_e))
        for c in cps:
            c.wait()
        pltpu.sync_copy(rows_v, eout.at[pl.ds(w * IPW + g * IPC, IPC)])
        return carry

    lax.fori_loop(0, NCH, chunk, 0)


def _sc_fc_body(x3, fc, fcg, idx_v, fcr_v, sem_f):
    w = lax.axis_index("s") * NC + lax.axis_index("c")
    pltpu.sync_copy(x3.at[pl.ds(w * RPW, RPW)], idx_v)

    def chunk(g, carry):
        cps = []
        for j in range(RPC):
            r = g * RPC + j
            cps.append(pltpu.async_copy(
                fc.at[idx_v.at[r]], fcr_v.at[pl.ds(r * 128, 128)], sem_f))
        for c in cps:
            c.wait()
        return carry

    lax.fori_loop(0, NCH, chunk, 0)
    pltpu.sync_copy(fcr_v, fcg.at[pl.ds(w * IPW, IPW)])


@functools.lru_cache(maxsize=None)
def _make_sc_emb():
    return pl.kernel(
        _sc_emb_body,
        out_type=jax.ShapeDtypeStruct((BF, D), jnp.float32),
        mesh=plsc.VectorSubcoreMesh(
            core_axis_name="c", subcore_axis_name="s",
            num_cores=NC, num_subcores=NS),
        scratch_types=[
            pltpu.VMEM((RPC, 128), jnp.int32),
            pltpu.VMEM((IPC, D), jnp.float32),
            pltpu.SemaphoreType.DMA,
        ],
        compiler_params=pltpu.CompilerParams(use_tc_tiling_on_sc=False),
    )


@functools.lru_cache(maxsize=None)
def _make_sc_fc():
    return pl.kernel(
        _sc_fc_body,
        out_type=jax.ShapeDtypeStruct((BF,), jnp.float32),
        mesh=plsc.VectorSubcoreMesh(
            core_axis_name="c", subcore_axis_name="s",
            num_cores=NC, num_subcores=NS),
        scratch_types=[
            pltpu.VMEM((RPW, 128), jnp.int32),
            pltpu.VMEM((IPW,), jnp.float32),
            pltpu.SemaphoreType.DMA,
        ],
        compiler_params=pltpu.CompilerParams(use_tc_tiling_on_sc=False),
    )


def _mlp_body(e_ref, fcg_ref, s_ref, w1_ref, b1_ref, w2_ref, b2_ref,
              w3_ref, b3_ref, w4_ref, c_ref, o_ref):
    e = e_ref[...]
    s = jnp.dot(e, s_ref[...], preferred_element_type=jnp.float32)
    q = jnp.dot(e * e, s_ref[...], preferred_element_type=jnp.float32)
    pair = 0.5 * jnp.sum(s * s - q, axis=1, keepdims=True)
    lin = jnp.sum(fcg_ref[...], axis=1, keepdims=True)
    h = jnp.maximum(
        jnp.dot(e, w1_ref[...], preferred_element_type=jnp.float32)
        + b1_ref[...], 0.0)
    h = jnp.maximum(
        jnp.dot(h, w2_ref[...], preferred_element_type=jnp.float32)
        + b2_ref[...], 0.0)
    h = jnp.maximum(
        jnp.dot(h, w3_ref[...], preferred_element_type=jnp.float32)
        + b3_ref[...], 0.0)
    mlp = jnp.sum(h * w4_ref[...], axis=1, keepdims=True)
    z = pair + lin + mlp + c_ref[0, 0]
    o_ref[...] = 1.0 / (1.0 + jnp.exp(-z))


def kernel(x, bias, fc, emb, W1, b1, W2, b2, W3, b3, W4, b4):
    x3 = x.reshape(BF // 128, 128)
    eflat = _make_sc_emb()(x3, emb)
    fcg = _make_sc_fc()(x3, fc[:, 0])
    e2 = eflat.reshape(B, ED)
    fcg2 = fcg.reshape(B, F)
    rows = jnp.arange(ED, dtype=jnp.int32)
    S = (rows[:, None] % D
         == jnp.arange(D, dtype=jnp.int32)[None, :]).astype(jnp.float32)
    c = (bias + b4).reshape(1, 1)

    bB = 1024
    grid = (B // bB,)
    out2 = pl.pallas_call(
        _mlp_body,
        grid=grid,
        in_specs=[
            pl.BlockSpec((bB, ED), lambda i: (i, 0)),
            pl.BlockSpec((bB, F), lambda i: (i, 0)),
            pl.BlockSpec((ED, D), lambda i: (0, 0)),
            pl.BlockSpec((ED, H1), lambda i: (0, 0)),
            pl.BlockSpec((1, H1), lambda i: (0, 0)),
            pl.BlockSpec((H1, H2), lambda i: (0, 0)),
            pl.BlockSpec((1, H2), lambda i: (0, 0)),
            pl.BlockSpec((H2, H3), lambda i: (0, 0)),
            pl.BlockSpec((1, H3), lambda i: (0, 0)),
            pl.BlockSpec((1, H3), lambda i: (0, 0)),
            pl.BlockSpec((1, 1), lambda i: (0, 0)),
        ],
        out_specs=pl.BlockSpec((bB, 1), lambda i: (i, 0)),
        out_shape=jax.ShapeDtypeStruct((B, 1), jnp.float32),
    )(e2, fcg2, S, W1, b1.reshape(1, H1), W2, b2.reshape(1, H2),
      W3, b3.reshape(1, H3), W4.reshape(1, H3), c)
    return out2.reshape(B)
